# baseline (speedup 1.0000x reference)
import functools

import jax
import jax.numpy as jnp
from jax import lax
from jax.experimental import pallas as pl
from jax.experimental.pallas import tpu as pltpu
from jax.experimental.pallas import tpu_sc as plsc

_N = 10000
_NBUF = 3
_E = 320000
_NC, _NS = 2, 16
_NW = _NC * _NS
_EPW = _E // _NW
_NP = 10240
_STRIPE = _NP // _NS


def _make_mp(D: int, chunk: int):
    nchunks = _EPW // chunk
    mesh = plsc.VectorSubcoreMesh(
        core_axis_name="c", subcore_axis_name="s",
        num_cores=_NC, num_subcores=_NS)

    @functools.partial(
        pl.kernel,
        mesh=mesh,
        out_type=jax.ShapeDtypeStruct((_NC * _NP, D), jnp.float32),
        scratch_types=[
            pltpu.VMEM((nchunks, chunk), jnp.int32),
            pltpu.VMEM((nchunks, chunk), jnp.int32),
            [pltpu.VMEM((chunk, D), jnp.float32) for _ in range(_NBUF)],
            [pltpu.SemaphoreType.DMA for _ in range(_NBUF)],
            [pltpu.SemaphoreType.DMA for _ in range(_NBUF)],
            pltpu.VMEM_SHARED((_NP, D), jnp.float32),
            pltpu.VMEM_SHARED((_NP, D), jnp.float32),
        ],
        compiler_params=pltpu.CompilerParams(use_tc_tiling_on_sc=False),
    )
    def mp(h_hbm, src_hbm, dst_hbm, zero_hbm, out_hbm,
           src_v, dst_v, bufs, gsem, ssem, h_sh, acc_sh):
        c = lax.axis_index("c")
        s = lax.axis_index("s")
        wid = c * _NS + s
        pltpu.sync_copy(src_hbm.at[wid], src_v)
        pltpu.sync_copy(dst_hbm.at[wid], dst_v)
        pltpu.sync_copy(h_hbm.at[pl.ds(s * _STRIPE, _STRIPE)],
                        h_sh.at[pl.ds(s * _STRIPE, _STRIPE)])
        pltpu.sync_copy(zero_hbm.at[pl.ds(s * _STRIPE, _STRIPE)],
                        acc_sh.at[pl.ds(s * _STRIPE, _STRIPE)])
        plsc.subcore_barrier()

        gather = [None] * _NBUF
        scatter = [None] * _NBUF
        for j in range(min(_NBUF, nchunks)):
            gather[j] = pltpu.async_copy(
                h_sh.at[src_v.at[j]], bufs[j], gsem[j])
        for j in range(nchunks):
            b = j % _NBUF
            gather[b].wait()
            scatter[b] = pltpu.async_copy(
                bufs[b], acc_sh.at[dst_v.at[j]], ssem[b], add=True)
            nj = j + _NBUF
            if nj < nchunks:
                scatter[b].wait()
                scatter[b] = None
                gather[b] = pltpu.async_copy(
                    h_sh.at[src_v.at[nj]], bufs[b], gsem[b])
        for d in scatter:
            if d is not None:
                d.wait()
        plsc.subcore_barrier()
        pltpu.sync_copy(acc_sh.at[pl.ds(s * _STRIPE, _STRIPE)],
                        out_hbm.at[pl.ds(c * _NP + s * _STRIPE, _STRIPE)])

    return mp


_CHUNK = 625
_NCHUNKS = _EPW // _CHUNK
_mp32 = _make_mp(32, _CHUNK)
_mp16 = _make_mp(16, _CHUNK)


def _mm1_body(x_ref, w_ref, o_ref):
    o_ref[:_N, :] = jnp.dot(x_ref[...], w_ref[...],
                            preferred_element_type=jnp.float32)
    o_ref[_N:, :] = jnp.zeros((_NP - _N, 32), jnp.float32)


def _mm2_body(p_ref, w_ref, o_ref):
    h = jax.nn.relu(p_ref[:_N, :] + p_ref[_NP:_NP + _N, :])
    o_ref[:_N, :] = jnp.dot(h, w_ref[...], preferred_element_type=jnp.float32)
    o_ref[_N:, :] = jnp.zeros((_NP - _N, 16), jnp.float32)


def _softmax_body(p_ref, o_ref):
    z = p_ref[:_N, :] + p_ref[_NP:_NP + _N, :]
    z = z - jnp.max(z, axis=-1, keepdims=True)
    e = jnp.exp(z)
    o_ref[...] = e / jnp.sum(e, axis=-1, keepdims=True)


def kernel(x, edge_index, W1, W2):
    src = edge_index[0].astype(jnp.int32).reshape(_NW, _NCHUNKS, _CHUNK)
    dst = edge_index[1].astype(jnp.int32).reshape(_NW, _NCHUNKS, _CHUNK)
    z32 = jnp.zeros((_NP, 32), jnp.float32)
    z16 = jnp.zeros((_NP, 16), jnp.float32)

    h1pre = pl.pallas_call(
        _mm1_body,
        out_shape=jax.ShapeDtypeStruct((_NP, 32), jnp.float32),
    )(x, W1)

    m1 = _mp32(h1pre, src, dst, z32)

    h2pre = pl.pallas_call(
        _mm2_body,
        out_shape=jax.ShapeDtypeStruct((_NP, 16), jnp.float32),
    )(m1, W2)

    m2 = _mp16(h2pre, src, dst, z16)

    out = pl.pallas_call(
        _softmax_body,
        out_shape=jax.ShapeDtypeStruct((_N, 16), jnp.float32),
    )(m2)
    return out


# speedup vs baseline: 1.0342x; 1.0342x over previous
import functools

import jax
import jax.numpy as jnp
from jax import lax
from jax.experimental import pallas as pl
from jax.experimental.pallas import tpu as pltpu
from jax.experimental.pallas import tpu_sc as plsc

_N = 10000
_NBUF = 3
_E = 320000
_NC, _NS = 2, 16
_NW = _NC * _NS
_EPW = _E // _NW
_NP = 10240
_STRIPE = _NP // _NS


def _make_mp(D: int, chunk: int):
    nchunks = _EPW // chunk
    mesh = plsc.VectorSubcoreMesh(
        core_axis_name="c", subcore_axis_name="s",
        num_cores=_NC, num_subcores=_NS)

    @functools.partial(
        pl.kernel,
        mesh=mesh,
        out_type=jax.ShapeDtypeStruct((_NC * _NP, D), jnp.float32),
        scratch_types=[
            pltpu.VMEM((nchunks, chunk), jnp.int32),
            pltpu.VMEM((nchunks, chunk), jnp.int32),
            [pltpu.VMEM((chunk, D), jnp.float32) for _ in range(_NBUF)],
            [pltpu.SemaphoreType.DMA for _ in range(_NBUF)],
            [pltpu.SemaphoreType.DMA for _ in range(_NBUF)],
            pltpu.VMEM_SHARED((_NP, D), jnp.float32),
        ],
        compiler_params=pltpu.CompilerParams(use_tc_tiling_on_sc=False),
    )
    def mp(h_hbm, src_hbm, dst_hbm, zero_hbm, out_hbm,
           src_v, dst_v, bufs, gsem, ssem, acc_sh):
        c = lax.axis_index("c")
        s = lax.axis_index("s")
        wid = c * _NS + s
        pltpu.sync_copy(src_hbm.at[wid], src_v)
        pltpu.sync_copy(dst_hbm.at[wid], dst_v)
        pltpu.sync_copy(zero_hbm.at[pl.ds(s * _STRIPE, _STRIPE)],
                        acc_sh.at[pl.ds(s * _STRIPE, _STRIPE)])
        plsc.subcore_barrier()

        gather = [None] * _NBUF
        scatter = [None] * _NBUF
        for j in range(min(_NBUF, nchunks)):
            gather[j] = pltpu.async_copy(
                h_hbm.at[src_v.at[j]], bufs[j], gsem[j])
        for j in range(nchunks):
            b = j % _NBUF
            gather[b].wait()
            scatter[b] = pltpu.async_copy(
                bufs[b], acc_sh.at[dst_v.at[j]], ssem[b], add=True)
            nj = j + _NBUF
            if nj < nchunks:
                scatter[b].wait()
                scatter[b] = None
                gather[b] = pltpu.async_copy(
                    h_hbm.at[src_v.at[nj]], bufs[b], gsem[b])
        for d in scatter:
            if d is not None:
                d.wait()
        plsc.subcore_barrier()
        pltpu.sync_copy(acc_sh.at[pl.ds(s * _STRIPE, _STRIPE)],
                        out_hbm.at[pl.ds(c * _NP + s * _STRIPE, _STRIPE)])

    return mp


_CHUNK = 625
_NCHUNKS = _EPW // _CHUNK
_mp32 = _make_mp(32, _CHUNK)
_mp16 = _make_mp(16, _CHUNK)


def _mm1_body(x_ref, w_ref, o_ref):
    o_ref[...] = jnp.dot(x_ref[...], w_ref[...],
                         preferred_element_type=jnp.float32)


def _mm2_body(p_ref, w_ref, o_ref):
    h = jax.nn.relu(p_ref[:_N, :] + p_ref[_NP:_NP + _N, :])
    o_ref[...] = jnp.dot(h, w_ref[...], preferred_element_type=jnp.float32)


def _softmax_body(p_ref, o_ref):
    z = p_ref[:_N, :] + p_ref[_NP:_NP + _N, :]
    z = z - jnp.max(z, axis=-1, keepdims=True)
    e = jnp.exp(z)
    o_ref[...] = e / jnp.sum(e, axis=-1, keepdims=True)


def kernel(x, edge_index, W1, W2):
    src = edge_index[0].astype(jnp.int32).reshape(_NW, _NCHUNKS, _CHUNK)
    dst = edge_index[1].astype(jnp.int32).reshape(_NW, _NCHUNKS, _CHUNK)
    z32 = jnp.zeros((_NP, 32), jnp.float32)
    z16 = jnp.zeros((_NP, 16), jnp.float32)

    h1pre = pl.pallas_call(
        _mm1_body,
        out_shape=jax.ShapeDtypeStruct((_N, 32), jnp.float32),
    )(x, W1)

    m1 = _mp32(h1pre, src, dst, z32)

    h2pre = pl.pallas_call(
        _mm2_body,
        out_shape=jax.ShapeDtypeStruct((_N, 16), jnp.float32),
    )(m1, W2)

    m2 = _mp16(h2pre, src, dst, z16)

    out = pl.pallas_call(
        _softmax_body,
        out_shape=jax.ShapeDtypeStruct((_N, 16), jnp.float32),
    )(m2)
    return out


# speedup vs baseline: 1.0828x; 1.0471x over previous
import functools

import jax
import jax.numpy as jnp
from jax import lax
from jax.experimental import pallas as pl
from jax.experimental.pallas import tpu as pltpu
from jax.experimental.pallas import tpu_sc as plsc

_N = 10000
_NBUF = 2
_E = 320000
_NC, _NS = 2, 16
_NW = _NC * _NS
_EP = 327680
_EPW = _EP // _NW
_NP = 10240
_STRIPE = _NP // _NS


def _make_mp(D: int, chunk: int):
    nchunks = _EPW // chunk
    bpw = _EPW // 128
    mesh = plsc.VectorSubcoreMesh(
        core_axis_name="c", subcore_axis_name="s",
        num_cores=_NC, num_subcores=_NS)

    @functools.partial(
        pl.kernel,
        mesh=mesh,
        out_type=jax.ShapeDtypeStruct((_NC * _NP, D), jnp.float32),
        scratch_types=[
            pltpu.VMEM((2 * bpw, 128), jnp.int32),
            pltpu.VMEM((nchunks, chunk), jnp.int32),
            pltpu.VMEM((nchunks, chunk), jnp.int32),
            [pltpu.VMEM((chunk, D), jnp.float32) for _ in range(_NBUF)],
            [pltpu.SemaphoreType.DMA for _ in range(_NBUF)],
            [pltpu.SemaphoreType.DMA for _ in range(_NBUF)],
            pltpu.VMEM_SHARED((_NP, D), jnp.float32),
        ],
        compiler_params=pltpu.CompilerParams(use_tc_tiling_on_sc=False),
    )
    def mp(h_hbm, e3_hbm, zero_hbm, out_hbm,
           e3_v, src_v, dst_v, bufs, gsem, ssem, acc_sh):
        c = lax.axis_index("c")
        s = lax.axis_index("s")
        wid = c * _NS + s
        pltpu.sync_copy(e3_hbm.at[pl.ds(wid * 2 * bpw, 2 * bpw)], e3_v)
        pltpu.sync_copy(zero_hbm.at[pl.ds(s * _STRIPE, _STRIPE)],
                        acc_sh.at[pl.ds(s * _STRIPE, _STRIPE)])
        bpc = chunk // 128
        for blk in range(bpw):
            r, col0 = blk // bpc, (blk % bpc) * 128
            for l in range(0, 128, 16):
                src_v[r, pl.ds(col0 + l, 16)] = e3_v[2 * blk, pl.ds(l, 16)]
                dst_v[r, pl.ds(col0 + l, 16)] = e3_v[2 * blk + 1, pl.ds(l, 16)]
        plsc.subcore_barrier()

        gather = [None] * _NBUF
        scatter = [None] * _NBUF
        for j in range(min(_NBUF, nchunks)):
            gather[j] = pltpu.async_copy(
                h_hbm.at[src_v.at[j]], bufs[j], gsem[j])
        for j in range(nchunks):
            b = j % _NBUF
            gather[b].wait()
            scatter[b] = pltpu.async_copy(
                bufs[b], acc_sh.at[dst_v.at[j]], ssem[b], add=True)
            nj = j + _NBUF
            if nj < nchunks:
                scatter[b].wait()
                scatter[b] = None
                gather[b] = pltpu.async_copy(
                    h_hbm.at[src_v.at[nj]], bufs[b], gsem[b])
        for d in scatter:
            if d is not None:
                d.wait()
        plsc.subcore_barrier()
        pltpu.sync_copy(acc_sh.at[pl.ds(s * _STRIPE, _STRIPE)],
                        out_hbm.at[pl.ds(c * _NP + s * _STRIPE, _STRIPE)])

    return mp


_CHUNK = 1024
_NCHUNKS = _EPW // _CHUNK
_mp32 = _make_mp(32, _CHUNK)
_mp16 = _make_mp(16, _CHUNK)


def _mm1_body(x_ref, w_ref, o_ref):
    o_ref[...] = jnp.dot(x_ref[...], w_ref[...],
                         preferred_element_type=jnp.float32)


def _mm2_body(p_ref, w_ref, o_ref):
    h = jax.nn.relu(p_ref[:_N, :] + p_ref[_NP:_NP + _N, :])
    o_ref[...] = jnp.dot(h, w_ref[...], preferred_element_type=jnp.float32)


def _softmax_body(p_ref, o_ref):
    z = p_ref[:_N, :] + p_ref[_NP:_NP + _N, :]
    z = z - jnp.max(z, axis=-1, keepdims=True)
    e = jnp.exp(z)
    o_ref[...] = e / jnp.sum(e, axis=-1, keepdims=True)


def kernel(x, edge_index, W1, W2):
    npad = _EP - _E
    pad = jnp.stack([jnp.arange(npad, dtype=jnp.int32) % _N,
                     _N + (jnp.arange(npad, dtype=jnp.int32) % (_NP - _N))])
    ep = jnp.concatenate([edge_index.astype(jnp.int32), pad], axis=1)
    e3 = ep.reshape(2, _EP // 128, 128).transpose(1, 0, 2).reshape(_EP // 64, 128)
    z32 = jnp.zeros((_NP, 32), jnp.float32)
    z16 = jnp.zeros((_NP, 16), jnp.float32)

    h1pre = pl.pallas_call(
        _mm1_body,
        out_shape=jax.ShapeDtypeStruct((_N, 32), jnp.float32),
    )(x, W1)

    m1 = _mp32(h1pre, e3, z32)

    h2pre = pl.pallas_call(
        _mm2_body,
        out_shape=jax.ShapeDtypeStruct((_N, 16), jnp.float32),
    )(m1, W2)

    m2 = _mp16(h2pre, e3, z16)

    out = pl.pallas_call(
        _softmax_body,
        out_shape=jax.ShapeDtypeStruct((_N, 16), jnp.float32),
    )(m2)
    return out


# speedup vs baseline: 1.0917x; 1.0082x over previous
import functools

import numpy as np

import jax
import jax.numpy as jnp
from jax import lax
from jax.experimental import pallas as pl
from jax.experimental.pallas import tpu as pltpu
from jax.experimental.pallas import tpu_sc as plsc

_N = 10000
_NBUF = 2
_E = 320000
_NC, _NS = 2, 16
_NW = _NC * _NS
_EP = 327680
_EPW = _EP // _NW
_NP = 10240
_STRIPE = _NP // _NS


def _make_mp(D: int, chunk: int):
    nchunks = _EPW // chunk
    bpw = _EPW // 128
    mesh = plsc.VectorSubcoreMesh(
        core_axis_name="c", subcore_axis_name="s",
        num_cores=_NC, num_subcores=_NS)

    @functools.partial(
        pl.kernel,
        mesh=mesh,
        out_type=jax.ShapeDtypeStruct((_NC * _NP, D), jnp.float32),
        scratch_types=[
            pltpu.VMEM((2 * bpw, 128), jnp.int32),
            pltpu.VMEM((nchunks, chunk), jnp.int32),
            pltpu.VMEM((nchunks, chunk), jnp.int32),
            [pltpu.VMEM((chunk, D), jnp.float32) for _ in range(_NBUF)],
            [pltpu.SemaphoreType.DMA for _ in range(_NBUF)],
            [pltpu.SemaphoreType.DMA for _ in range(_NBUF)],
            pltpu.VMEM_SHARED((_NP, D), jnp.float32),
        ],
        compiler_params=pltpu.CompilerParams(use_tc_tiling_on_sc=False),
    )
    def mp(h_hbm, e3_hbm, zero_hbm, out_hbm,
           e3_v, src_v, dst_v, bufs, gsem, ssem, acc_sh):
        c = lax.axis_index("c")
        s = lax.axis_index("s")
        wid = c * _NS + s
        pltpu.sync_copy(e3_hbm.at[pl.ds(wid * 2 * bpw, 2 * bpw)], e3_v)
        pltpu.sync_copy(zero_hbm.at[pl.ds(s * _STRIPE, _STRIPE)],
                        acc_sh.at[pl.ds(s * _STRIPE, _STRIPE)])
        bpc = chunk // 128
        for blk in range(bpw):
            r, col0 = blk // bpc, (blk % bpc) * 128
            for l in range(0, 128, 16):
                src_v[r, pl.ds(col0 + l, 16)] = e3_v[2 * blk, pl.ds(l, 16)]
                dst_v[r, pl.ds(col0 + l, 16)] = e3_v[2 * blk + 1, pl.ds(l, 16)]
        plsc.subcore_barrier()

        gather = [None] * _NBUF
        scatter = [None] * _NBUF
        for j in range(min(_NBUF, nchunks)):
            gather[j] = pltpu.async_copy(
                h_hbm.at[src_v.at[j]], bufs[j], gsem[j])
        for j in range(nchunks):
            b = j % _NBUF
            gather[b].wait()
            scatter[b] = pltpu.async_copy(
                bufs[b], acc_sh.at[dst_v.at[j]], ssem[b], add=True)
            nj = j + _NBUF
            if nj < nchunks:
                scatter[b].wait()
                scatter[b] = None
                gather[b] = pltpu.async_copy(
                    h_hbm.at[src_v.at[nj]], bufs[b], gsem[b])
        for d in scatter:
            if d is not None:
                d.wait()
        plsc.subcore_barrier()
        pltpu.sync_copy(acc_sh.at[pl.ds(s * _STRIPE, _STRIPE)],
                        out_hbm.at[pl.ds(c * _NP + s * _STRIPE, _STRIPE)])

    return mp


_CHUNK = 1024
_NCHUNKS = _EPW // _CHUNK
_mp32 = _make_mp(32, _CHUNK)
_mp16 = _make_mp(16, _CHUNK)


def _mm1_body(x_ref, w_ref, o_ref):
    o_ref[...] = jnp.dot(x_ref[...], w_ref[...],
                         preferred_element_type=jnp.float32)


def _mm2_body(p_ref, w_ref, o_ref):
    h = jax.nn.relu(p_ref[:_N, :] + p_ref[_NP:_NP + _N, :])
    o_ref[...] = jnp.dot(h, w_ref[...], preferred_element_type=jnp.float32)


def _softmax_body(p_ref, o_ref):
    z = p_ref[:_N, :] + p_ref[_NP:_NP + _N, :]
    z = z - jnp.max(z, axis=-1, keepdims=True)
    e = jnp.exp(z)
    o_ref[...] = e / jnp.sum(e, axis=-1, keepdims=True)


def kernel(x, edge_index, W1, W2):
    npad = _EP - _E
    pad = np.stack([np.arange(npad, dtype=np.int32) % _N,
                    _N + (np.arange(npad, dtype=np.int32) % (_NP - _N))])
    ep = jnp.concatenate([edge_index.astype(jnp.int32), pad], axis=1)
    e3 = ep.reshape(2, _EP // 128, 128).transpose(1, 0, 2).reshape(_EP // 64, 128)
    z32 = jnp.zeros((_NP, 32), jnp.float32)
    z16 = jnp.zeros((_NP, 16), jnp.float32)

    h1pre = pl.pallas_call(
        _mm1_body,
        out_shape=jax.ShapeDtypeStruct((_N, 32), jnp.float32),
    )(x, W1)

    m1 = _mp32(h1pre, e3, z32)

    h2pre = pl.pallas_call(
        _mm2_body,
        out_shape=jax.ShapeDtypeStruct((_N, 16), jnp.float32),
    )(m1, W2)

    m2 = _mp16(h2pre, e3, z16)

    out = pl.pallas_call(
        _softmax_body,
        out_shape=jax.ShapeDtypeStruct((_N, 16), jnp.float32),
    )(m2)
    return out


# speedup vs baseline: 1.1197x; 1.0257x over previous
import functools

import numpy as np

import jax
import jax.numpy as jnp
from jax import lax
from jax.experimental import pallas as pl
from jax.experimental.pallas import tpu as pltpu
from jax.experimental.pallas import tpu_sc as plsc

_N = 10000
_NBUF = 3
_E = 320000
_NC, _NS = 2, 16
_NW = _NC * _NS
_EP = 327680
_EPW = _EP // _NW
_NP = 10240
_STRIPE = _NP // _NS


def _make_mp(D: int, chunk: int):
    nchunks = _EPW // chunk
    bpw = _EPW // 128
    mesh = plsc.VectorSubcoreMesh(
        core_axis_name="c", subcore_axis_name="s",
        num_cores=_NC, num_subcores=_NS)

    @functools.partial(
        pl.kernel,
        mesh=mesh,
        out_type=jax.ShapeDtypeStruct((_NC * _NP, D), jnp.float32),
        scratch_types=[
            pltpu.VMEM((2 * bpw, 128), jnp.int32),
            pltpu.VMEM((nchunks, chunk), jnp.int32),
            pltpu.VMEM((nchunks, chunk), jnp.int32),
            [pltpu.VMEM((chunk, D), jnp.float32) for _ in range(_NBUF)],
            [pltpu.SemaphoreType.DMA for _ in range(_NBUF)],
            [pltpu.SemaphoreType.DMA for _ in range(_NBUF)],
            pltpu.VMEM_SHARED((_NP, D), jnp.float32),
        ],
        compiler_params=pltpu.CompilerParams(use_tc_tiling_on_sc=False),
    )
    def mp(h_hbm, e3_hbm, zero_hbm, out_hbm,
           e3_v, src_v, dst_v, bufs, gsem, ssem, acc_sh):
        c = lax.axis_index("c")
        s = lax.axis_index("s")
        wid = c * _NS + s
        pltpu.sync_copy(e3_hbm.at[pl.ds(wid * 2 * bpw, 2 * bpw)], e3_v)
        pltpu.sync_copy(zero_hbm.at[pl.ds(s * _STRIPE, _STRIPE)],
                        acc_sh.at[pl.ds(s * _STRIPE, _STRIPE)])
        bpc = chunk // 128
        for blk in range(bpw):
            r, col0 = blk // bpc, (blk % bpc) * 128
            for l in range(0, 128, 16):
                src_v[r, pl.ds(col0 + l, 16)] = e3_v[2 * blk, pl.ds(l, 16)]
                dst_v[r, pl.ds(col0 + l, 16)] = e3_v[2 * blk + 1, pl.ds(l, 16)]
        plsc.subcore_barrier()

        gather = [None] * _NBUF
        scatter = [None] * _NBUF
        for j in range(min(_NBUF, nchunks)):
            gather[j] = pltpu.async_copy(
                h_hbm.at[src_v.at[j]], bufs[j], gsem[j])
        for j in range(nchunks):
            b = j % _NBUF
            gather[b].wait()
            scatter[b] = pltpu.async_copy(
                bufs[b], acc_sh.at[dst_v.at[j]], ssem[b], add=True)
            nj = j + _NBUF
            if nj < nchunks:
                scatter[b].wait()
                scatter[b] = None
                gather[b] = pltpu.async_copy(
                    h_hbm.at[src_v.at[nj]], bufs[b], gsem[b])
        for d in scatter:
            if d is not None:
                d.wait()
        plsc.subcore_barrier()
        pltpu.sync_copy(acc_sh.at[pl.ds(s * _STRIPE, _STRIPE)],
                        out_hbm.at[pl.ds(c * _NP + s * _STRIPE, _STRIPE)])

    return mp


_CHUNK = 640
_NCHUNKS = _EPW // _CHUNK
_mp32 = _make_mp(32, _CHUNK)
_mp16 = _make_mp(16, _CHUNK)


def _mm1_body(x_ref, w_ref, o_ref):
    o_ref[...] = jnp.dot(x_ref[...], w_ref[...],
                         preferred_element_type=jnp.float32)


def _mm2_body(p_ref, w_ref, o_ref):
    h = jax.nn.relu(p_ref[:_N, :] + p_ref[_NP:_NP + _N, :])
    o_ref[...] = jnp.dot(h, w_ref[...], preferred_element_type=jnp.float32)


def _softmax_body(p_ref, o_ref):
    z = p_ref[:_N, :] + p_ref[_NP:_NP + _N, :]
    z = z - jnp.max(z, axis=-1, keepdims=True)
    e = jnp.exp(z)
    o_ref[...] = e / jnp.sum(e, axis=-1, keepdims=True)


def kernel(x, edge_index, W1, W2):
    npad = _EP - _E
    pad = np.stack([np.arange(npad, dtype=np.int32) % _N,
                    _N + (np.arange(npad, dtype=np.int32) % (_NP - _N))])
    ep = jnp.concatenate([edge_index.astype(jnp.int32), pad], axis=1)
    e3 = ep.reshape(2, _EP // 128, 128).transpose(1, 0, 2).reshape(_EP // 64, 128)
    z32 = jnp.zeros((_NP, 32), jnp.float32)
    z16 = jnp.zeros((_NP, 16), jnp.float32)

    h1pre = pl.pallas_call(
        _mm1_body,
        out_shape=jax.ShapeDtypeStruct((_N, 32), jnp.float32),
    )(x, W1)

    m1 = _mp32(h1pre, e3, z32)

    h2pre = pl.pallas_call(
        _mm2_body,
        out_shape=jax.ShapeDtypeStruct((_N, 16), jnp.float32),
    )(m1, W2)

    m2 = _mp16(h2pre, e3, z16)

    out = pl.pallas_call(
        _softmax_body,
        out_shape=jax.ShapeDtypeStruct((_N, 16), jnp.float32),
    )(m2)
    return out
